# Initial kernel scaffold; baseline (speedup 1.0000x reference)
import jax
import jax.numpy as jnp
from jax.experimental import pallas as pl


def kernel(positions, positional_embeddings):
    raise NotImplementedError("write your pallas kernel here")


# speedup vs baseline: 2.3102x; 2.3102x over previous
import functools

import jax
import jax.numpy as jnp
from jax import lax
from jax.experimental import pallas as pl
from jax.experimental.pallas import tpu as pltpu
from jax.experimental.pallas import tpu_sc as plsc

N_CTX = 8192
N_STATE = 1024


_ROWS_PER_BLOCK = 1024


def _normalize_body(emb_ref, out_ref):
    x = emb_ref[...]
    norm = jnp.sqrt(jnp.sum(x * x, axis=-1, keepdims=True))
    out_ref[...] = x / jnp.maximum(norm, 1e-12)


def _normalize_table(table):
    n_rows, d = table.shape
    grid = (n_rows // _ROWS_PER_BLOCK,)
    return pl.pallas_call(
        _normalize_body,
        grid=grid,
        in_specs=[pl.BlockSpec((_ROWS_PER_BLOCK, d), lambda i: (i, 0))],
        out_specs=pl.BlockSpec((_ROWS_PER_BLOCK, d), lambda i: (i, 0)),
        out_shape=jax.ShapeDtypeStruct((n_rows, d), table.dtype),
    )(table)


_NC = 2
_NS = 16
_NW = _NC * _NS
_CHUNK = 32


def _make_gather(B, D):
    b_per_w = B // _NW
    n_chunks = b_per_w // _CHUNK
    n_pairs = n_chunks // 2
    mesh = plsc.VectorSubcoreMesh(core_axis_name="c", subcore_axis_name="s")

    @functools.partial(
        pl.kernel,
        mesh=mesh,
        out_type=jax.ShapeDtypeStruct((B, D), jnp.float32),
        scratch_types=[
            pltpu.VMEM((b_per_w,), jnp.int32),
            pltpu.VMEM((_CHUNK, D), jnp.float32),
            pltpu.VMEM((_CHUNK, D), jnp.float32),
            pltpu.SemaphoreType.DMA,
            pltpu.SemaphoreType.DMA,
        ],
    )
    def gather(table_hbm, idx_hbm, out_hbm, idx_v, rows0, rows1, sem0, sem1):
        wid = lax.axis_index("s") * _NC + lax.axis_index("c")
        base = wid * b_per_w
        pltpu.sync_copy(idx_hbm.at[pl.ds(base, b_per_w)], idx_v)

        bufs = (rows0, rows1)
        sems = (sem0, sem1)

        def start(g, slot):
            return pltpu.async_copy(
                table_hbm.at[idx_v.at[pl.ds(g * _CHUNK, _CHUNK)]],
                bufs[slot], sems[slot])

        def drain(slot):
            pltpu.make_async_copy(
                table_hbm.at[pl.ds(0, _CHUNK)], bufs[slot], sems[slot]).wait()

        def flush(g, slot):
            pltpu.sync_copy(
                bufs[slot], out_hbm.at[pl.ds(base + g * _CHUNK, _CHUNK)])

        start(0, 0)

        def body(i, _):
            g = 2 * i
            cp_odd = start(g + 1, 1)
            drain(0)
            flush(g, 0)

            @pl.when(i < n_pairs - 1)
            def _():
                start(g + 2, 0)

            cp_odd.wait()
            flush(g + 1, 1)
            return 0

        lax.fori_loop(0, n_pairs, body, 0)

    return gather


def kernel(positions, positional_embeddings):
    bsz, seq = positions.shape
    n_rows, d = positional_embeddings.shape
    normed = _normalize_table(positional_embeddings)
    idx = positions.reshape(-1).astype(jnp.int32)
    out = _make_gather(bsz * seq, d)(normed, idx)
    return out.reshape(bsz, seq, d)
